# baseline (speedup 1.0000x reference)
import functools

import jax
import jax.numpy as jnp
from jax import lax
from jax.experimental import pallas as pl
from jax.experimental.pallas import tpu as pltpu
from jax.experimental.pallas import tpu_sc as plsc

B, T, N, D = 1, 4, 10000, 128
HEADS, HEAD_DIM = 4, 32
E = 160000
BT = B * T

NS = 16
CH = 80
Q = CH // 4
EDGES_PER_TILE = E // NS
NCHUNK = EDGES_PER_TILE // CH
NPAD = 10240
ROWS_PER_TILE = NPAD // NS
ROW_BLOCK = 1000
AW = 16


def _perm(v, idx):
    return lax.gather(
        v, idx[:, None],
        dimension_numbers=lax.GatherDimensionNumbers(
            offset_dims=(), collapsed_slice_dims=(0,), start_index_map=(0,)),
        slice_sizes=(1,),
        mode=lax.GatherScatterMode.PROMISE_IN_BOUNDS)


def _proj_body(x_ref, w_ref, a_ref, h_ref, al_ref):
    h = jnp.dot(x_ref[0], w_ref[...], preferred_element_type=jnp.float32)
    h_ref[0] = h
    al_ref[0] = jnp.dot(h, a_ref[...], preferred_element_type=jnp.float32)


def _project(x3, W, A):
    nblk = N // ROW_BLOCK
    return pl.pallas_call(
        _proj_body,
        grid=(BT, nblk),
        in_specs=[
            pl.BlockSpec((1, ROW_BLOCK, D), lambda t, i: (t, i, 0)),
            pl.BlockSpec((D, D), lambda t, i: (0, 0)),
            pl.BlockSpec((D, AW), lambda t, i: (0, 0)),
        ],
        out_specs=[
            pl.BlockSpec((1, ROW_BLOCK, D), lambda t, i: (t, i, 0)),
            pl.BlockSpec((1, ROW_BLOCK, AW), lambda t, i: (t, i, 0)),
        ],
        out_shape=[
            jax.ShapeDtypeStruct((BT, N, D), jnp.float32),
            jax.ShapeDtypeStruct((BT, N, AW), jnp.float32),
        ],
    )(x3, W, A)


def _sc_body(h_hbm, alpha_hbm, src_hbm, dst_hbm, ew_hbm, zrow_hbm, zden_hbm,
             agg_out, den_out,
             agg_sh, den_sh, src_v, dst_v, ew_v, asw_v, adw_v, exw_v, cf_v,
             rows_v, sem_row, sem_al, sem_sc):
    c = lax.axis_index("c")
    s = lax.axis_index("s")
    row0 = s * ROWS_PER_TILE
    ebase = s * EDGES_PER_TILE
    lane = lax.iota(jnp.int32, 16)

    def idx_load(i, b):
        base = ebase + i * CH
        pltpu.sync_copy(src_hbm.at[pl.ds(base, CH)], src_v[b])
        pltpu.sync_copy(dst_hbm.at[pl.ds(base, CH)], dst_v[b])
        pltpu.sync_copy(ew_hbm.at[pl.ds(base, CH)], ew_v[b])

    def issue_gathers(t, b):
        pltpu.async_copy(h_hbm.at[t].at[src_v[b]], rows_v[b], sem_row[b])
        pltpu.async_copy(alpha_hbm.at[t].at[src_v[b]], asw_v[b], sem_al[b])
        pltpu.async_copy(alpha_hbm.at[t].at[dst_v[b]], adw_v[b], sem_al[b])

    def wait_al(t, b):
        pltpu.make_async_copy(
            alpha_hbm.at[t].at[src_v[b]], asw_v[b], sem_al[b]).wait()
        pltpu.make_async_copy(
            alpha_hbm.at[t].at[dst_v[b]], adw_v[b], sem_al[b]).wait()

    def wait_row(t, b):
        pltpu.make_async_copy(
            h_hbm.at[t].at[src_v[b]], rows_v[b], sem_row[b]).wait()

    def issue_scatter(b):
        pltpu.async_copy(rows_v[b], agg_sh.at[dst_v[b]], sem_sc[b], add=True)
        pltpu.async_copy(exw_v[b], den_sh.at[dst_v[b]], sem_sc[b], add=True)

    def drain_scatter(b):
        pltpu.make_async_copy(rows_v[b], agg_sh.at[dst_v[b]], sem_sc[b]).wait()
        pltpu.make_async_copy(exw_v[b], den_sh.at[dst_v[b]], sem_sc[b]).wait()

    def compute_excf(b):
        for q in range(Q):
            e0 = 4 * q
            sv = None
            dv = None
            for r in range(4):
                rs = asw_v[b][e0 + r, :]
                rd = adw_v[b][e0 + r, :]
                ps = _perm(rs, (lane + (16 - 4 * r)) & 15)
                pd = _perm(rd, (lane + (20 - 4 * r)) & 15)
                if r == 0:
                    sv, dv = ps, pd
                else:
                    sel = lane >= (4 * r)
                    sv = jnp.where(sel, ps, sv)
                    dv = jnp.where(sel, pd, dv)
            e = sv + dv
            e = jnp.where(e >= 0.0, e, 0.2 * e)
            ex = jnp.exp(e)
            cb = (e0 // 16) * 16
            ewc = ew_v[b][pl.ds(cb, 16)]
            ewp = _perm(ewc, (lane >> 2) + (e0 - cb))
            cf_v[b][pl.ds(16 * q, 16)] = ex * ewp
            for r in range(4):
                exw_v[b][e0 + r, :] = _perm(ex, (lane & 3) + 4 * r)

    def scale_rows(b):
        def group_body(q4, carry2):
            for u in range(4):
                q2 = q4 * 4 + u
                cfc = cf_v[b][pl.ds(q2 * 16, 16)]
                for r in range(4):
                    e_i = q2 * 4 + r
                    for hh in range(HEADS):
                        spl = _perm(cfc, lane * 0 + (4 * r + hh))
                        for half in range(2):
                            off = hh * HEAD_DIM + half * 16
                            rows_v[b][e_i, pl.ds(off, 16)] = (
                                rows_v[b][e_i, pl.ds(off, 16)] * spl)
            return carry2

        lax.fori_loop(0, Q // 4, group_body, 0)

    for tt in range(BT // 2):
        t = c * (BT // 2) + tt

        pltpu.sync_copy(zrow_hbm.at[pl.ds(row0, ROWS_PER_TILE)],
                        agg_sh.at[pl.ds(row0, ROWS_PER_TILE)])
        pltpu.sync_copy(zden_hbm.at[pl.ds(row0, ROWS_PER_TILE)],
                        den_sh.at[pl.ds(row0, ROWS_PER_TILE)])
        plsc.subcore_barrier()

        idx_load(0, 0)
        issue_gathers(t, 0)

        def pair_body(m, carry):
            i0 = 2 * m
            for b in (0, 1):
                i = i0 + b
                bo = 1 - b
                wait_al(t, b)
                compute_excf(b)
                if b == 0:
                    @pl.when(m > 0)
                    def _():
                        drain_scatter(bo)
                else:
                    drain_scatter(bo)
                idx_load(i + 1, bo)
                issue_gathers(t, bo)
                wait_row(t, b)
                scale_rows(b)
                issue_scatter(b)
            return carry

        lax.fori_loop(0, (NCHUNK - 1) // 2, pair_body, 0)

        wait_al(t, 0)
        compute_excf(0)
        wait_row(t, 0)
        scale_rows(0)
        issue_scatter(0)
        drain_scatter(0)
        drain_scatter(1)
        plsc.subcore_barrier()

        pltpu.sync_copy(agg_sh.at[pl.ds(row0, ROWS_PER_TILE)],
                        agg_out.at[t].at[pl.ds(row0, ROWS_PER_TILE)])
        pltpu.sync_copy(den_sh.at[pl.ds(row0, ROWS_PER_TILE)],
                        den_out.at[t].at[pl.ds(row0, ROWS_PER_TILE)])
        plsc.subcore_barrier()


_sc_edge_pass = functools.partial(
    pl.kernel,
    mesh=plsc.VectorSubcoreMesh(core_axis_name="c", subcore_axis_name="s"),
    compiler_params=pltpu.CompilerParams(use_tc_tiling_on_sc=False),
    out_type=[
        jax.ShapeDtypeStruct((BT, NPAD, D), jnp.float32),
        jax.ShapeDtypeStruct((BT, NPAD, AW), jnp.float32),
    ],
    scratch_types=[
        pltpu.VMEM_SHARED((NPAD, D), jnp.float32),
        pltpu.VMEM_SHARED((NPAD, AW), jnp.float32),
        (pltpu.VMEM((CH,), jnp.int32),) * 2,
        (pltpu.VMEM((CH,), jnp.int32),) * 2,
        (pltpu.VMEM((CH,), jnp.float32),) * 2,
        (pltpu.VMEM((CH, AW), jnp.float32),) * 2,
        (pltpu.VMEM((CH, AW), jnp.float32),) * 2,
        (pltpu.VMEM((CH, AW), jnp.float32),) * 2,
        (pltpu.VMEM((HEADS * CH,), jnp.float32),) * 2,
        (pltpu.VMEM((CH, D), jnp.float32),) * 2,
        (pltpu.SemaphoreType.DMA,) * 2,
        (pltpu.SemaphoreType.DMA,) * 2,
        (pltpu.SemaphoreType.DMA,) * 2,
    ],
)(_sc_body)


def _post_body(agg_ref, den_ref, bias_ref, gamma_ref, beta_ref, out_ref):
    agg = agg_ref[0]
    den = den_ref[0][:, :HEADS]
    col = lax.broadcasted_iota(jnp.int32, (HEADS, D), 1) // HEAD_DIM
    row = lax.broadcasted_iota(jnp.int32, (HEADS, D), 0)
    rep = (col == row).astype(jnp.float32)
    den_rep = jnp.dot(den, rep, preferred_element_type=jnp.float32)
    out = agg / (den_rep + 1e-16) + bias_ref[...]
    mu = jnp.mean(out, axis=-1, keepdims=True)
    var = jnp.mean((out - mu) ** 2, axis=-1, keepdims=True)
    out = (out - mu) * lax.rsqrt(var + 1e-5) * gamma_ref[...] + beta_ref[...]
    out_ref[0] = jnp.where(out > 0.0, out, jnp.exp(out) - 1.0)


def _postprocess(agg, den, bias, gamma, beta):
    nblk = N // ROW_BLOCK
    return pl.pallas_call(
        _post_body,
        grid=(BT, nblk),
        in_specs=[
            pl.BlockSpec((1, ROW_BLOCK, D), lambda t, i: (t, i, 0)),
            pl.BlockSpec((1, ROW_BLOCK, AW), lambda t, i: (t, i, 0)),
            pl.BlockSpec((1, D), lambda t, i: (0, 0)),
            pl.BlockSpec((1, D), lambda t, i: (0, 0)),
            pl.BlockSpec((1, D), lambda t, i: (0, 0)),
        ],
        out_specs=pl.BlockSpec((1, ROW_BLOCK, D), lambda t, i: (t, i, 0)),
        out_shape=jax.ShapeDtypeStruct((BT, N, D), jnp.float32),
    )(agg, den, bias.reshape(1, D), gamma.reshape(1, D), beta.reshape(1, D))


def kernel(x, edge_index, edge_weight, W, a_src, a_dst, bias, gamma, beta):
    x3 = x.reshape(BT, N, D).astype(jnp.float32)
    head_of_col = jnp.repeat(jnp.arange(HEADS), HEAD_DIM)
    one_hot = jax.nn.one_hot(head_of_col, HEADS, dtype=jnp.float32)
    A = jnp.concatenate(
        [one_hot * a_src.reshape(D)[:, None],
         one_hot * a_dst.reshape(D)[:, None],
         jnp.zeros((D, AW - 2 * HEADS), jnp.float32)], axis=1)

    h3, alpha3 = _project(x3, W.astype(jnp.float32), A)

    src = edge_index[0].astype(jnp.int32)
    dst = edge_index[1].astype(jnp.int32)
    ew = edge_weight.astype(jnp.float32)
    zrow = jnp.zeros((NPAD, D), jnp.float32)
    zden = jnp.zeros((NPAD, AW), jnp.float32)

    agg, den = _sc_edge_pass(h3, alpha3, src, dst, ew, zrow, zden)

    out = _postprocess(agg, den, bias, gamma, beta)
    return out.reshape(B, T, N, D)


# speedup vs baseline: 1.5482x; 1.5482x over previous
import functools

import jax
import jax.numpy as jnp
from jax import lax
from jax.experimental import pallas as pl
from jax.experimental.pallas import tpu as pltpu
from jax.experimental.pallas import tpu_sc as plsc

B, T, N, D = 1, 4, 10000, 128
HEADS, HEAD_DIM = 4, 32
E = 160000
BT = B * T

NS = 16
CH = 80
Q = CH // 4
EDGES_PER_TILE = E // NS
NCHUNK = EDGES_PER_TILE // CH
LSEG = 25
NPAD = 10240
ROWS_PER_TILE = NPAD // NS
ROW_BLOCK = 1000
AW = 16


def _perm(v, idx):
    return lax.gather(
        v, idx[:, None],
        dimension_numbers=lax.GatherDimensionNumbers(
            offset_dims=(), collapsed_slice_dims=(0,), start_index_map=(0,)),
        slice_sizes=(1,),
        mode=lax.GatherScatterMode.PROMISE_IN_BOUNDS)


def _proj_body(x_ref, w_ref, a_ref, h_ref, al_ref):
    h = jnp.dot(x_ref[0], w_ref[...], preferred_element_type=jnp.float32)
    h_ref[0] = h
    al_ref[0] = jnp.dot(h, a_ref[...], preferred_element_type=jnp.float32)


def _project(x3, W, A):
    nblk = N // ROW_BLOCK
    return pl.pallas_call(
        _proj_body,
        grid=(BT, nblk),
        in_specs=[
            pl.BlockSpec((1, ROW_BLOCK, D), lambda t, i: (t, i, 0)),
            pl.BlockSpec((D, D), lambda t, i: (0, 0)),
            pl.BlockSpec((D, AW), lambda t, i: (0, 0)),
        ],
        out_specs=[
            pl.BlockSpec((1, ROW_BLOCK, D), lambda t, i: (t, i, 0)),
            pl.BlockSpec((1, ROW_BLOCK, AW), lambda t, i: (t, i, 0)),
        ],
        out_shape=[
            jax.ShapeDtypeStruct((BT, N, D), jnp.float32),
            jax.ShapeDtypeStruct((BT, N, AW), jnp.float32),
        ],
    )(x3, W, A)


def _sc_body(h_hbm, alpha_hbm, src_hbm, dst_hbm, ew_hbm, zrow_hbm, zden_hbm,
             agg_out, den_out,
             agg_sh, den_sh, src_big, dst_big, ew_big, asw_v, adw_v, exw_v,
             cf_v, rows_v, sem_row, sem_al, sem_sc):
    c = lax.axis_index("c")
    s = lax.axis_index("s")
    row0 = s * ROWS_PER_TILE
    crow0 = s * NCHUNK
    lane = lax.iota(jnp.int32, 16)

    def issue_gathers(t, i, b):
        pltpu.async_copy(h_hbm.at[t].at[src_big.at[i]], rows_v[b], sem_row[b])
        pltpu.async_copy(alpha_hbm.at[t].at[src_big.at[i]], asw_v[b],
                         sem_al[b])
        pltpu.async_copy(alpha_hbm.at[t].at[dst_big.at[i]], adw_v[b],
                         sem_al[b])

    def wait_al(t, b):
        pltpu.make_async_copy(
            alpha_hbm.at[t].at[src_big.at[0]], asw_v[b], sem_al[b]).wait()
        pltpu.make_async_copy(
            alpha_hbm.at[t].at[dst_big.at[0]], adw_v[b], sem_al[b]).wait()

    def wait_row(t, b):
        pltpu.make_async_copy(
            h_hbm.at[t].at[src_big.at[0]], rows_v[b], sem_row[b]).wait()

    def issue_scatter(i, b):
        pltpu.async_copy(rows_v[b], agg_sh.at[dst_big.at[i]], sem_sc[b],
                         add=True)
        pltpu.async_copy(exw_v[b], den_sh.at[dst_big.at[i]], sem_sc[b],
                         add=True)

    def drain_scatter(b):
        pltpu.make_async_copy(rows_v[b], agg_sh.at[dst_big.at[0]],
                              sem_sc[b]).wait()
        pltpu.make_async_copy(exw_v[b], den_sh.at[dst_big.at[0]],
                              sem_sc[b]).wait()

    def compute_excf(i, b):
        for q in range(Q):
            e0 = 4 * q
            sv = None
            dv = None
            for r in range(4):
                rs = asw_v[b][e0 + r, :]
                rd = adw_v[b][e0 + r, :]
                ps = _perm(rs, (lane + (16 - 4 * r)) & 15)
                pd = _perm(rd, (lane + (20 - 4 * r)) & 15)
                if r == 0:
                    sv, dv = ps, pd
                else:
                    sel = lane >= (4 * r)
                    sv = jnp.where(sel, ps, sv)
                    dv = jnp.where(sel, pd, dv)
            e = sv + dv
            e = jnp.where(e >= 0.0, e, 0.2 * e)
            ex = jnp.exp(e)
            cb = (e0 // 16) * 16
            ewc = ew_big[i, pl.ds(cb, 16)]
            ewp = _perm(ewc, (lane >> 2) + (e0 - cb))
            cf_v[b][pl.ds(16 * q, 16)] = ex * ewp
            for r in range(4):
                exw_v[b][e0 + r, :] = _perm(ex, (lane & 3) + 4 * r)

    def scale_rows(b):
        def group_body(q4, carry2):
            for u in range(4):
                q2 = q4 * 4 + u
                cfc = cf_v[b][pl.ds(q2 * 16, 16)]
                for r in range(4):
                    e_i = q2 * 4 + r
                    for hh in range(HEADS):
                        spl = _perm(cfc, lane * 0 + (4 * r + hh))
                        for half in range(2):
                            off = hh * HEAD_DIM + half * 16
                            rows_v[b][e_i, pl.ds(off, 16)] = (
                                rows_v[b][e_i, pl.ds(off, 16)] * spl)
            return carry2

        lax.fori_loop(0, Q // 4, group_body, 0)

    for tt in range(BT // 2):
        t = c * (BT // 2) + tt

        pltpu.sync_copy(zrow_hbm.at[pl.ds(row0, ROWS_PER_TILE)],
                        agg_sh.at[pl.ds(row0, ROWS_PER_TILE)])
        pltpu.sync_copy(zden_hbm.at[pl.ds(row0, ROWS_PER_TILE)],
                        den_sh.at[pl.ds(row0, ROWS_PER_TILE)])
        plsc.subcore_barrier()

        def seg_body(g, carry0):
            cbase = crow0 + g * LSEG
            pltpu.sync_copy(src_hbm.at[pl.ds(cbase, LSEG)], src_big)
            pltpu.sync_copy(dst_hbm.at[pl.ds(cbase, LSEG)], dst_big)
            pltpu.sync_copy(ew_hbm.at[pl.ds(cbase, LSEG)], ew_big)

            issue_gathers(t, 0, 0)

            def pair_body(m, carry):
                i0 = 2 * m
                for b in (0, 1):
                    i = i0 + b
                    bo = 1 - b
                    wait_al(t, b)
                    compute_excf(i, b)
                    if b == 0:
                        @pl.when(m > 0)
                        def _():
                            drain_scatter(bo)
                    else:
                        drain_scatter(bo)
                    issue_gathers(t, i + 1, bo)
                    wait_row(t, b)
                    scale_rows(b)
                    issue_scatter(i, b)
                return carry

            lax.fori_loop(0, (LSEG - 1) // 2, pair_body, 0)

            wait_al(t, 0)
            compute_excf(LSEG - 1, 0)
            wait_row(t, 0)
            scale_rows(0)
            issue_scatter(LSEG - 1, 0)
            drain_scatter(0)
            drain_scatter(1)
            return carry0

        lax.fori_loop(0, NCHUNK // LSEG, seg_body, 0)
        plsc.subcore_barrier()

        pltpu.sync_copy(agg_sh.at[pl.ds(row0, ROWS_PER_TILE)],
                        agg_out.at[t].at[pl.ds(row0, ROWS_PER_TILE)])
        pltpu.sync_copy(den_sh.at[pl.ds(row0, ROWS_PER_TILE)],
                        den_out.at[t].at[pl.ds(row0, ROWS_PER_TILE)])
        plsc.subcore_barrier()


_sc_edge_pass = functools.partial(
    pl.kernel,
    mesh=plsc.VectorSubcoreMesh(core_axis_name="c", subcore_axis_name="s"),
    compiler_params=pltpu.CompilerParams(use_tc_tiling_on_sc=False),
    out_type=[
        jax.ShapeDtypeStruct((BT, NPAD, D), jnp.float32),
        jax.ShapeDtypeStruct((BT, NPAD, AW), jnp.float32),
    ],
    scratch_types=[
        pltpu.VMEM_SHARED((NPAD, D), jnp.float32),
        pltpu.VMEM_SHARED((NPAD, AW), jnp.float32),
        pltpu.VMEM((LSEG, CH), jnp.int32),
        pltpu.VMEM((LSEG, CH), jnp.int32),
        pltpu.VMEM((LSEG, CH), jnp.float32),
        (pltpu.VMEM((CH, AW), jnp.float32),) * 2,
        (pltpu.VMEM((CH, AW), jnp.float32),) * 2,
        (pltpu.VMEM((CH, AW), jnp.float32),) * 2,
        (pltpu.VMEM((HEADS * CH,), jnp.float32),) * 2,
        (pltpu.VMEM((CH, D), jnp.float32),) * 2,
        (pltpu.SemaphoreType.DMA,) * 2,
        (pltpu.SemaphoreType.DMA,) * 2,
        (pltpu.SemaphoreType.DMA,) * 2,
    ],
)(_sc_body)


def _post_body(agg_ref, den_ref, bias_ref, gamma_ref, beta_ref, out_ref):
    agg = agg_ref[0]
    den = den_ref[0][:, :HEADS]
    col = lax.broadcasted_iota(jnp.int32, (HEADS, D), 1) // HEAD_DIM
    row = lax.broadcasted_iota(jnp.int32, (HEADS, D), 0)
    rep = (col == row).astype(jnp.float32)
    den_rep = jnp.dot(den, rep, preferred_element_type=jnp.float32)
    out = agg / (den_rep + 1e-16) + bias_ref[...]
    mu = jnp.mean(out, axis=-1, keepdims=True)
    var = jnp.mean((out - mu) ** 2, axis=-1, keepdims=True)
    out = (out - mu) * lax.rsqrt(var + 1e-5) * gamma_ref[...] + beta_ref[...]
    out_ref[0] = jnp.where(out > 0.0, out, jnp.exp(out) - 1.0)


def _postprocess(agg, den, bias, gamma, beta):
    nblk = N // ROW_BLOCK
    return pl.pallas_call(
        _post_body,
        grid=(BT, nblk),
        in_specs=[
            pl.BlockSpec((1, ROW_BLOCK, D), lambda t, i: (t, i, 0)),
            pl.BlockSpec((1, ROW_BLOCK, AW), lambda t, i: (t, i, 0)),
            pl.BlockSpec((1, D), lambda t, i: (0, 0)),
            pl.BlockSpec((1, D), lambda t, i: (0, 0)),
            pl.BlockSpec((1, D), lambda t, i: (0, 0)),
        ],
        out_specs=pl.BlockSpec((1, ROW_BLOCK, D), lambda t, i: (t, i, 0)),
        out_shape=jax.ShapeDtypeStruct((BT, N, D), jnp.float32),
    )(agg, den, bias.reshape(1, D), gamma.reshape(1, D), beta.reshape(1, D))


def kernel(x, edge_index, edge_weight, W, a_src, a_dst, bias, gamma, beta):
    x3 = x.reshape(BT, N, D).astype(jnp.float32)
    head_of_col = jnp.repeat(jnp.arange(HEADS), HEAD_DIM)
    one_hot = jax.nn.one_hot(head_of_col, HEADS, dtype=jnp.float32)
    A = jnp.concatenate(
        [one_hot * a_src.reshape(D)[:, None],
         one_hot * a_dst.reshape(D)[:, None],
         jnp.zeros((D, AW - 2 * HEADS), jnp.float32)], axis=1)

    h3, alpha3 = _project(x3, W.astype(jnp.float32), A)

    src = edge_index[0].astype(jnp.int32).reshape(E // CH, CH)
    dst = edge_index[1].astype(jnp.int32).reshape(E // CH, CH)
    ew = edge_weight.astype(jnp.float32).reshape(E // CH, CH)
    zrow = jnp.zeros((NPAD, D), jnp.float32)
    zden = jnp.zeros((NPAD, AW), jnp.float32)

    agg, den = _sc_edge_pass(h3, alpha3, src, dst, ew, zrow, zden)

    out = _postprocess(agg, den, bias, gamma, beta)
    return out.reshape(B, T, N, D)
